# baseline (speedup 1.0000x reference)
import functools

import jax
import jax.numpy as jnp
from jax import lax
from jax.experimental import pallas as pl
from jax.experimental.pallas import tpu as pltpu
from jax.experimental.pallas import tpu_sc as plsc

N = 16 * 512 * 512
NORM = float(512 * 512 * 16)
MIN_KEPT = 100000
START = N - MIN_KEPT

NW = 32
PER = N // NW
C = 8192
NCH = PER // C

NB1 = 2048
NB2 = 2048
NB3 = 512

_INV = 1.0 / NORM


def _wid():
    return lax.axis_index("s") * 2 + lax.axis_index("c")


def _zero_hist(ref, nwords):
    zero16 = jnp.zeros((16,), ref.dtype)

    def body(k, _):
        ref[pl.ds(k * 16, 16)] = zero16
        return 0

    lax.fori_loop(0, nwords // 16, body, 0)


def _stream(wid, hbm_refs, bufs0, bufs1, sem0, sem1, compute, init):
    def start(ci, bufs, sem):
        base = wid * PER + ci * C
        return [pltpu.async_copy(a.at[pl.ds(base, C)], b, sem)
                for a, b in zip(hbm_refs, bufs)]

    slots = (bufs0, bufs1)
    sems = (sem0, sem1)
    carry = init
    handles = {0: start(0, slots[0], sems[0])}
    for ci in range(NCH):
        if ci + 1 < NCH:
            s = (ci + 1) % 2
            handles[ci + 1] = start(ci + 1, slots[s], sems[s])
        for h in handles.pop(ci):
            h.wait()
        carry = compute(slots[ci % 2], carry)
    return carry


def _consts():
    lane = lax.iota(jnp.int32, 16)
    ones_i = jnp.ones((16,), jnp.int32)
    zero_i = jnp.zeros((16,), jnp.int32)
    zero_f = jnp.zeros((16,), jnp.float32)
    return lane, ones_i, zero_i, zero_f


def _build(interpret=False):
    _mesh = plsc.VectorSubcoreMesh(
        core_axis_name="c", subcore_axis_name="s",
        num_cores=2, num_subcores=16)

    @functools.partial(
        pl.kernel,
        out_type=(
            jax.ShapeDtypeStruct((NW * NB1 * 16,), jnp.int32),
            jax.ShapeDtypeStruct((NW * NB1 * 16,), jnp.int32),
        ),
        mesh=_mesh,
        scratch_types=[
            pltpu.VMEM((NB1 * 16,), jnp.int32),
            pltpu.VMEM((NB1 * 16,), jnp.int32),
            pltpu.VMEM((C,), jnp.float32),
            pltpu.VMEM((C,), jnp.float32),
            pltpu.VMEM((C,), jnp.float32),
            pltpu.VMEM((C,), jnp.float32),
            pltpu.SemaphoreType.DMA,
            pltpu.SemaphoreType.DMA,
        ],
        compiler_params=pltpu.CompilerParams(needs_layout_passes=False),
        interpret=interpret,
    )
    def pass1(p_hbm, t_hbm, ha_hbm, hb_hbm,
              hist_a, hist_b, pb0, tb0, pb1, tb1, sem0, sem1):
        lane, ones_i, zero_i, zero_f = _consts()
        wid = _wid()
        _zero_hist(hist_a, NB1 * 16)
        _zero_hist(hist_b, NB1 * 16)
        hists = (hist_a, hist_b)

        def compute(bufs, carry):
            pbuf, tbuf = bufs

            def inner(j, _):
                for k in range(4):
                    o = pl.ds(j * 64 + k * 16, 16)
                    d = pbuf[o] - tbuf[o]
                    l = (d * d) * _INV
                    bits = lax.bitcast_convert_type(l, jnp.int32)
                    idx = ((bits >> 20) << 4) + lane
                    plsc.addupdate_scatter(hists[k % 2], [idx], ones_i)
                return 0

            lax.fori_loop(0, C // 64, inner, 0)
            return carry

        _stream(wid, [p_hbm, t_hbm], [pb0, tb0], [pb1, tb1],
                sem0, sem1, compute, 0)
        pltpu.sync_copy(hist_a, ha_hbm.at[pl.ds(wid * NB1 * 16, NB1 * 16)])
        pltpu.sync_copy(hist_b, hb_hbm.at[pl.ds(wid * NB1 * 16, NB1 * 16)])

    @functools.partial(
        pl.kernel,
        out_type=(
            jax.ShapeDtypeStruct((NW * NB2 * 16,), jnp.int32),
            jax.ShapeDtypeStruct((NW * NB2 * 16,), jnp.int32),
            jax.ShapeDtypeStruct((NW * 16,), jnp.int32),
            jax.ShapeDtypeStruct((NW * 16,), jnp.float32),
        ),
        mesh=_mesh,
        scratch_types=[
            pltpu.VMEM((NB2 * 16,), jnp.int32),
            pltpu.VMEM((NB2 * 16,), jnp.int32),
            pltpu.VMEM((C,), jnp.float32),
            pltpu.VMEM((C,), jnp.float32),
            pltpu.VMEM((C,), jnp.float32),
            pltpu.VMEM((C,), jnp.float32),
            pltpu.VMEM((C,), jnp.float32),
            pltpu.VMEM((C,), jnp.float32),
            pltpu.VMEM((16,), jnp.int32),
            pltpu.VMEM((16,), jnp.int32),
            pltpu.VMEM((16,), jnp.float32),
            pltpu.SemaphoreType.DMA,
            pltpu.SemaphoreType.DMA,
        ],
        compiler_params=pltpu.CompilerParams(needs_layout_passes=False),
        interpret=interpret,
    )
    def pass2(p_hbm, t_hbm, w_hbm, d1_hbm, ha_hbm, hb_hbm, cnt_hbm, ws_hbm,
              hist_a, hist_b, pb0, tb0, wb0, pb1, tb1, wb1, d1_v, cnt_v, ws_v,
              sem0, sem1):
        lane, ones_i, zero_i, zero_f = _consts()
        wid = _wid()
        _zero_hist(hist_a, NB2 * 16)
        _zero_hist(hist_b, NB2 * 16)
        hists = (hist_a, hist_b)
        pltpu.sync_copy(d1_hbm, d1_v)
        d1 = d1_v[...]

        def compute(bufs, carry):
            pbuf, tbuf, wbuf = bufs

            def inner(j, c2):
                cnt0, ws0, cnt1, ws1 = c2
                accs = [(cnt0, ws0), (cnt1, ws1)]
                for k in range(4):
                    o = pl.ds(j * 64 + k * 16, 16)
                    pv = pbuf[o]
                    tv = tbuf[o]
                    wv = wbuf[o]
                    d = pv - tv
                    d2 = d * d
                    wl = (wv * d2) * _INV
                    bits = lax.bitcast_convert_type(d2 * _INV, jnp.int32)
                    hi = bits >> 20
                    mid = (bits >> 9) & 0x7FF
                    idx = (mid << 4) + lane
                    plsc.addupdate_scatter(hists[k % 2], [idx], ones_i,
                                           mask=hi == d1)
                    m_ab = hi > d1
                    cnt, ws = accs[k % 2]
                    accs[k % 2] = (cnt + jnp.where(m_ab, ones_i, zero_i),
                                   ws + jnp.where(m_ab, wl, zero_f))
                return (accs[0][0], accs[0][1], accs[1][0], accs[1][1])

            return lax.fori_loop(0, C // 64, inner, carry)

        cnt0, ws0, cnt1, ws1 = _stream(
            wid, [p_hbm, t_hbm, w_hbm], [pb0, tb0, wb0], [pb1, tb1, wb1],
            sem0, sem1, compute, (zero_i, zero_f, zero_i, zero_f))
        cnt_v[...] = cnt0 + cnt1
        ws_v[...] = ws0 + ws1
        pltpu.sync_copy(hist_a, ha_hbm.at[pl.ds(wid * NB2 * 16, NB2 * 16)])
        pltpu.sync_copy(hist_b, hb_hbm.at[pl.ds(wid * NB2 * 16, NB2 * 16)])
        pltpu.sync_copy(cnt_v, cnt_hbm.at[pl.ds(wid * 16, 16)])
        pltpu.sync_copy(ws_v, ws_hbm.at[pl.ds(wid * 16, 16)])

    @functools.partial(
        pl.kernel,
        out_type=(
            jax.ShapeDtypeStruct((NW * NB3 * 16,), jnp.int32),
            jax.ShapeDtypeStruct((NW * NB3 * 16,), jnp.int32),
            jax.ShapeDtypeStruct((NW * NB3 * 16,), jnp.float32),
            jax.ShapeDtypeStruct((NW * NB3 * 16,), jnp.float32),
            jax.ShapeDtypeStruct((NW * 16,), jnp.int32),
            jax.ShapeDtypeStruct((NW * 16,), jnp.float32),
        ),
        mesh=_mesh,
        scratch_types=[
            pltpu.VMEM((NB3 * 16,), jnp.int32),
            pltpu.VMEM((NB3 * 16,), jnp.int32),
            pltpu.VMEM((NB3 * 16,), jnp.float32),
            pltpu.VMEM((NB3 * 16,), jnp.float32),
            pltpu.VMEM((C,), jnp.float32),
            pltpu.VMEM((C,), jnp.float32),
            pltpu.VMEM((C,), jnp.float32),
            pltpu.VMEM((C,), jnp.float32),
            pltpu.VMEM((C,), jnp.float32),
            pltpu.VMEM((C,), jnp.float32),
            pltpu.VMEM((16,), jnp.int32),
            pltpu.VMEM((16,), jnp.int32),
            pltpu.VMEM((16,), jnp.int32),
            pltpu.VMEM((16,), jnp.float32),
            pltpu.SemaphoreType.DMA,
            pltpu.SemaphoreType.DMA,
        ],
        compiler_params=pltpu.CompilerParams(needs_layout_passes=False),
        interpret=interpret,
    )
    def pass3(p_hbm, t_hbm, w_hbm, t22_hbm, hiend_hbm,
              ha_hbm, hb_hbm, wa_hbm, wb_hbm, cnt_hbm, ws_hbm,
              hist_a, hist_b, whist_a, whist_b, pb0, tb0, wb0, pb1, tb1, wb1,
              t22_v, hiend_v, cnt_v, ws_v, sem0, sem1):
        lane, ones_i, zero_i, zero_f = _consts()
        wid = _wid()
        _zero_hist(hist_a, NB3 * 16)
        _zero_hist(hist_b, NB3 * 16)
        _zero_hist(whist_a, NB3 * 16)
        _zero_hist(whist_b, NB3 * 16)
        hists = (hist_a, hist_b)
        whists = (whist_a, whist_b)
        pltpu.sync_copy(t22_hbm, t22_v)
        pltpu.sync_copy(hiend_hbm, hiend_v)
        t22 = t22_v[...]
        hiend = hiend_v[...]

        def compute(bufs, carry):
            pbuf, tbuf, wbuf = bufs

            def inner(j, c2):
                cnt0, ws0, cnt1, ws1 = c2
                accs = [(cnt0, ws0), (cnt1, ws1)]
                for k in range(4):
                    o = pl.ds(j * 64 + k * 16, 16)
                    pv = pbuf[o]
                    tv = tbuf[o]
                    wv = wbuf[o]
                    d = pv - tv
                    d2 = d * d
                    wl = (wv * d2) * _INV
                    bits = lax.bitcast_convert_type(d2 * _INV, jnp.int32)
                    pfx = bits >> 9
                    m_in = pfx == t22
                    idx = ((bits & 0x1FF) << 4) + lane
                    plsc.addupdate_scatter(hists[k % 2], [idx], ones_i,
                                           mask=m_in)
                    plsc.addupdate_scatter(whists[k % 2], [idx], wl,
                                           mask=m_in)
                    m_ab = (pfx > t22) & (pfx <= hiend)
                    cnt, ws = accs[k % 2]
                    accs[k % 2] = (cnt + jnp.where(m_ab, ones_i, zero_i),
                                   ws + jnp.where(m_ab, wl, zero_f))
                return (accs[0][0], accs[0][1], accs[1][0], accs[1][1])

            return lax.fori_loop(0, C // 64, inner, carry)

        cnt0, ws0, cnt1, ws1 = _stream(
            wid, [p_hbm, t_hbm, w_hbm], [pb0, tb0, wb0], [pb1, tb1, wb1],
            sem0, sem1, compute, (zero_i, zero_f, zero_i, zero_f))
        cnt_v[...] = cnt0 + cnt1
        ws_v[...] = ws0 + ws1
        pltpu.sync_copy(hist_a, ha_hbm.at[pl.ds(wid * NB3 * 16, NB3 * 16)])
        pltpu.sync_copy(hist_b, hb_hbm.at[pl.ds(wid * NB3 * 16, NB3 * 16)])
        pltpu.sync_copy(whist_a, wa_hbm.at[pl.ds(wid * NB3 * 16, NB3 * 16)])
        pltpu.sync_copy(whist_b, wb_hbm.at[pl.ds(wid * NB3 * 16, NB3 * 16)])
        pltpu.sync_copy(cnt_v, cnt_hbm.at[pl.ds(wid * 16, 16)])
        pltpu.sync_copy(ws_v, ws_hbm.at[pl.ds(wid * 16, 16)])

    return pass1, pass2, pass3


_PASSES = None


def _get_passes():
    global _PASSES
    if _PASSES is None:
        _PASSES = _build()
    return _PASSES


def kernel(predict, target, weight):
    _pass1, _pass2, _pass3 = _get_passes()
    p = predict.reshape(-1)
    t = target.reshape(-1)
    w = weight.reshape(-1)

    h1a, h1b = _pass1(p, t)
    h1 = (h1a.reshape(NW, NB1, 16).sum(axis=(0, 2))
          + h1b.reshape(NW, NB1, 16).sum(axis=(0, 2)))
    cum1 = jnp.cumsum(h1)
    d1 = jnp.sum((cum1 <= START).astype(jnp.int32))
    r1 = START - (cum1[d1] - h1[d1])

    d1v = jnp.full((16,), d1, jnp.int32)
    h2a, h2b, cnt2, ws2 = _pass2(p, t, w, d1v)
    h2 = (h2a.reshape(NW, NB2, 16).sum(axis=(0, 2))
          + h2b.reshape(NW, NB2, 16).sum(axis=(0, 2)))
    cum2 = jnp.cumsum(h2)
    d2 = jnp.sum((cum2 <= r1).astype(jnp.int32))
    r2 = r1 - (cum2[d2] - h2[d2])

    t22 = d1 * NB2 + d2
    t22v = jnp.full((16,), t22, jnp.int32)
    hiendv = jnp.full((16,), d1 * NB2 + (NB2 - 1), jnp.int32)
    h3a, h3b, wh3a, wh3b, cnt3, ws3 = _pass3(p, t, w, t22v, hiendv)
    h3 = (h3a.reshape(NW, NB3, 16).sum(axis=(0, 2))
          + h3b.reshape(NW, NB3, 16).sum(axis=(0, 2)))
    wh3 = (wh3a.reshape(NW, NB3, 16).sum(axis=(0, 2))
           + wh3b.reshape(NW, NB3, 16).sum(axis=(0, 2)))
    cum3 = jnp.cumsum(h3)
    d3 = jnp.sum((cum3 <= r2).astype(jnp.int32))

    bins = jnp.arange(NB3)
    cnt_in = jnp.sum(jnp.where(bins > d3, h3, 0))
    ws_in = jnp.sum(jnp.where(bins > d3, wh3, 0.0))

    sel_cnt = cnt2.sum() + cnt3.sum() + cnt_in
    sel_sum = ws2.sum() + ws3.sum() + ws_in
    return sel_sum / jnp.maximum(sel_cnt, 1).astype(jnp.float32)
